# baseline (speedup 1.0000x reference)
import functools

import numpy as np
import jax
import jax.numpy as jnp
from jax import lax
from jax.experimental import pallas as pl
from jax.experimental.pallas import tpu as pltpu
from jax.experimental.pallas import tpu_sc as plsc

_B, _L, _V, _H, _O = 4096, 50, 100000, 128, 10000
_NCHUNK = 4
_BC = _B // _NCHUNK
_NC, _NS = 2, 16
_NW = _NC * _NS
_BPW = _BC // _NW
_CB = 2
_ROWS = _CB * _L
_NCH = _BPW // _CB
_LANES = 16
_NV = _H // _LANES


def _accum_stream(rows, acc_v, ch):
    for b in range(_CB):
        base = b * _L
        init = tuple(rows[base, pl.ds(j * _LANES, _LANES)] for j in range(_NV))

        @pl.loop(1, _L, init_carry=init, unroll=7)
        def _sum(r, accs):
            return tuple(
                accs[j] + rows[base + r, pl.ds(j * _LANES, _LANES)]
                for j in range(_NV)
            )

        accs = _sum
        for j in range(_NV):
            acc_v[ch * _CB + b, pl.ds(j * _LANES, _LANES)] = accs[j]


def _sc_pool_body(ids_hbm, emb_hbm, prev_hbm, out_hbm,
                  ids_v, rows0, rows1, acc_v, sem0, sem1):
    del prev_hbm
    c = lax.axis_index("c")
    s = lax.axis_index("s")
    w = c * _NS + s

    pltpu.sync_copy(ids_hbm.at[w], ids_v)

    pltpu.async_copy(emb_hbm.at[ids_v.at[0]], rows0, sem0)

    @pl.loop(0, _NCH, step=2)
    def _streams(ch):
        pltpu.async_copy(emb_hbm.at[ids_v.at[ch + 1]], rows1, sem1)
        pltpu.make_async_copy(emb_hbm.at[ids_v.at[ch]], rows0, sem0).wait()
        _accum_stream(rows0, acc_v, ch)

        @pl.when(ch + 2 < _NCH)
        def _():
            pltpu.async_copy(emb_hbm.at[ids_v.at[ch + 2]], rows0, sem0)

        pltpu.make_async_copy(emb_hbm.at[ids_v.at[ch + 1]], rows1, sem1).wait()
        _accum_stream(rows1, acc_v, ch + 1)

    inv = jnp.full((_LANES,), 1.0 / _L, jnp.float32)
    zero = jnp.zeros((_LANES,), jnp.float32)

    @pl.loop(0, _BPW)
    def _act(i):
        for j in range(_NV):
            v = acc_v[i, pl.ds(j * _LANES, _LANES)]
            acc_v[i, pl.ds(j * _LANES, _LANES)] = jnp.maximum(v * inv, zero)

    pltpu.sync_copy(acc_v, out_hbm.at[w])


_sc_pool = pl.kernel(
    _sc_pool_body,
    out_type=jax.ShapeDtypeStruct((_NW, _BPW, _H), jnp.float32),
    mesh=plsc.VectorSubcoreMesh(core_axis_name="c", subcore_axis_name="s"),
    scratch_types=[
        pltpu.VMEM((_NCH, _ROWS), jnp.int32),
        pltpu.VMEM((_ROWS, _H), jnp.float32),
        pltpu.VMEM((_ROWS, _H), jnp.float32),
        pltpu.VMEM((_BPW, _H), jnp.float32),
        pltpu.SemaphoreType.DMA,
        pltpu.SemaphoreType.DMA,
    ],
)


_MB = 512
_GM = _BC // _MB


def _mm_body_first(x_ref, w_ref, b_ref, o_ref):
    o_ref[...] = lax.dot_general(
        x_ref[...], w_ref[...], (((1,), (1,)), ((), ())),
        preferred_element_type=jnp.float32,
    ) + b_ref[...]


def _mm_body_rest(x_ref, w_ref, b_ref, prev_ref, o_ref):
    del prev_ref
    o_ref[...] = lax.dot_general(
        x_ref[...], w_ref[...], (((1,), (1,)), ((), ())),
        preferred_element_type=jnp.float32,
    ) + b_ref[...]


def _mm_chunk(k, x, W, b2, prev):
    out_spec = pl.BlockSpec((_MB, _O), lambda i, k=k: (k * _GM + i, 0))
    in_specs = [
        pl.BlockSpec((_MB, _H), lambda i: (i, 0)),
        pl.BlockSpec((_O, _H), lambda i: (0, 0)),
        pl.BlockSpec((1, _O), lambda i: (0, 0)),
    ]
    if k == 0:
        return pl.pallas_call(
            _mm_body_first,
            grid=(_GM,),
            in_specs=in_specs,
            out_specs=out_spec,
            out_shape=jax.ShapeDtypeStruct((_B, _O), jnp.float32),
        )(x, W, b2)
    return pl.pallas_call(
        _mm_body_rest,
        grid=(_GM,),
        in_specs=in_specs + [pl.BlockSpec(memory_space=pl.ANY)],
        out_specs=out_spec,
        out_shape=jax.ShapeDtypeStruct((_B, _O), jnp.float32),
        input_output_aliases={3: 0},
    )(x, W, b2, prev)


@jax.jit
def _impl(input_ids, emb, W, b):
    b2 = b.reshape(1, _O)
    ids = input_ids.reshape(_NCHUNK, _NW, _NCH, _ROWS)
    prev_pooled = jnp.zeros((_NW, _BPW, _H), jnp.float32)
    out = None
    for k in range(_NCHUNK):
        pooled = _sc_pool(ids[k], emb, prev_pooled)
        out = _mm_chunk(k, pooled.reshape(_BC, _H), W, b2, out)
        prev_pooled = pooled
    return out


def kernel(input_ids, token_type_ids, attention_mask, emb, W, b):
    return _impl(input_ids, emb, W, b)


# speedup vs baseline: 1.0849x; 1.0849x over previous
import functools

import numpy as np
import jax
import jax.numpy as jnp
from jax import lax
from jax.experimental import pallas as pl
from jax.experimental.pallas import tpu as pltpu
from jax.experimental.pallas import tpu_sc as plsc

_B, _L, _V, _H, _O = 4096, 50, 100000, 128, 10000
_NCHUNK = 4
_BC = _B // _NCHUNK
_NC, _NS = 2, 16
_NW = _NC * _NS
_BPW = _BC // _NW
_CB = 2
_ROWS = _CB * _L
_NCH = _BPW // _CB
_LANES = 16
_NV = _H // _LANES


def _accum_stream(rows, acc_v, ch):
    for b in range(_CB):
        base = b * _L
        init = tuple(rows[base, pl.ds(j * _LANES, _LANES)] for j in range(_NV))

        @pl.loop(1, _L, init_carry=init, unroll=7)
        def _sum(r, accs):
            return tuple(
                accs[j] + rows[base + r, pl.ds(j * _LANES, _LANES)]
                for j in range(_NV)
            )

        accs = _sum
        for j in range(_NV):
            acc_v[ch * _CB + b, pl.ds(j * _LANES, _LANES)] = accs[j]


def _sc_pool_body(ids_hbm, emb_hbm, out_hbm,
                  ids_v, rows0, rows1, acc_v, sem0, sem1):
    c = lax.axis_index("c")
    s = lax.axis_index("s")
    w = c * _NS + s

    pltpu.sync_copy(ids_hbm.at[w], ids_v)

    pltpu.async_copy(emb_hbm.at[ids_v.at[0]], rows0, sem0)

    @pl.loop(0, _NCH, step=2)
    def _streams(ch):
        pltpu.async_copy(emb_hbm.at[ids_v.at[ch + 1]], rows1, sem1)
        pltpu.make_async_copy(emb_hbm.at[ids_v.at[ch]], rows0, sem0).wait()
        _accum_stream(rows0, acc_v, ch)

        @pl.when(ch + 2 < _NCH)
        def _():
            pltpu.async_copy(emb_hbm.at[ids_v.at[ch + 2]], rows0, sem0)

        pltpu.make_async_copy(emb_hbm.at[ids_v.at[ch + 1]], rows1, sem1).wait()
        _accum_stream(rows1, acc_v, ch + 1)

    inv = jnp.full((_LANES,), 1.0 / _L, jnp.float32)
    zero = jnp.zeros((_LANES,), jnp.float32)

    @pl.loop(0, _BPW)
    def _act(i):
        for j in range(_NV):
            v = acc_v[i, pl.ds(j * _LANES, _LANES)]
            acc_v[i, pl.ds(j * _LANES, _LANES)] = jnp.maximum(v * inv, zero)

    pltpu.sync_copy(acc_v, out_hbm.at[w])


_sc_pool = pl.kernel(
    _sc_pool_body,
    out_type=jax.ShapeDtypeStruct((_NW, _BPW, _H), jnp.float32),
    mesh=plsc.VectorSubcoreMesh(core_axis_name="c", subcore_axis_name="s"),
    scratch_types=[
        pltpu.VMEM((_NCH, _ROWS), jnp.int32),
        pltpu.VMEM((_ROWS, _H), jnp.float32),
        pltpu.VMEM((_ROWS, _H), jnp.float32),
        pltpu.VMEM((_BPW, _H), jnp.float32),
        pltpu.SemaphoreType.DMA,
        pltpu.SemaphoreType.DMA,
    ],
)


_MB = 512
_GM = _BC // _MB


def _mm_body_first(x_ref, w_ref, b_ref, o_ref):
    o_ref[...] = lax.dot_general(
        x_ref[...], w_ref[...], (((1,), (1,)), ((), ())),
        preferred_element_type=jnp.float32,
    ) + b_ref[...]


def _mm_body_rest(x_ref, w_ref, b_ref, prev_ref, o_ref):
    del prev_ref
    o_ref[...] = lax.dot_general(
        x_ref[...], w_ref[...], (((1,), (1,)), ((), ())),
        preferred_element_type=jnp.float32,
    ) + b_ref[...]


def _mm_chunk(k, x, W, b2, prev):
    out_spec = pl.BlockSpec((_MB, _O), lambda i, k=k: (k * _GM + i, 0))
    in_specs = [
        pl.BlockSpec((_MB, _H), lambda i: (i, 0)),
        pl.BlockSpec((_O, _H), lambda i: (0, 0)),
        pl.BlockSpec((1, _O), lambda i: (0, 0)),
    ]
    if k == 0:
        return pl.pallas_call(
            _mm_body_first,
            grid=(_GM,),
            in_specs=in_specs,
            out_specs=out_spec,
            out_shape=jax.ShapeDtypeStruct((_B, _O), jnp.float32),
        )(x, W, b2)
    return pl.pallas_call(
        _mm_body_rest,
        grid=(_GM,),
        in_specs=in_specs + [pl.BlockSpec(memory_space=pl.ANY)],
        out_specs=out_spec,
        out_shape=jax.ShapeDtypeStruct((_B, _O), jnp.float32),
        input_output_aliases={3: 0},
    )(x, W, b2, prev)


@jax.jit
def _impl(input_ids, emb, W, b):
    b2 = b.reshape(1, _O)
    ids = input_ids.reshape(_NCHUNK, _NW, _NCH, _ROWS)
    out = None
    for k in range(_NCHUNK):
        pooled = _sc_pool(ids[k], emb)
        out = _mm_chunk(k, pooled.reshape(_BC, _H), W, b2, out)
    return out


def kernel(input_ids, token_type_ids, attention_mask, emb, W, b):
    return _impl(input_ids, emb, W, b)


# speedup vs baseline: 1.0855x; 1.0005x over previous
import functools

import numpy as np
import jax
import jax.numpy as jnp
from jax import lax
from jax.experimental import pallas as pl
from jax.experimental.pallas import tpu as pltpu
from jax.experimental.pallas import tpu_sc as plsc

_B, _L, _V, _H, _O = 4096, 50, 100000, 128, 10000
_NCHUNK = 4
_BC = _B // _NCHUNK
_NC, _NS = 2, 16
_NW = _NC * _NS
_BPW = _BC // _NW
_CB = 2
_ROWS = _CB * _L
_NCH = _BPW // _CB
_LANES = 16
_NV = _H // _LANES


def _accum_stream(rows, acc_v, ch):
    for b in range(_CB):
        base = b * _L
        init = tuple(rows[base, pl.ds(j * _LANES, _LANES)] for j in range(_NV))

        @pl.loop(1, _L, init_carry=init, unroll=7)
        def _sum(r, accs):
            return tuple(
                accs[j] + rows[base + r, pl.ds(j * _LANES, _LANES)]
                for j in range(_NV)
            )

        accs = _sum
        for j in range(_NV):
            acc_v[ch * _CB + b, pl.ds(j * _LANES, _LANES)] = accs[j]


def _sc_pool_body(ids_hbm, emb_hbm, out_hbm,
                  ids_v, rows0, rows1, acc_v, sem0, sem1):
    c = lax.axis_index("c")
    s = lax.axis_index("s")
    w = c * _NS + s

    pltpu.sync_copy(ids_hbm.at[w], ids_v)

    pltpu.async_copy(emb_hbm.at[ids_v.at[0]], rows0, sem0)

    @pl.loop(0, _NCH, step=2)
    def _streams(ch):
        pltpu.async_copy(emb_hbm.at[ids_v.at[ch + 1]], rows1, sem1)
        pltpu.make_async_copy(emb_hbm.at[ids_v.at[ch]], rows0, sem0).wait()
        _accum_stream(rows0, acc_v, ch)

        @pl.when(ch + 2 < _NCH)
        def _():
            pltpu.async_copy(emb_hbm.at[ids_v.at[ch + 2]], rows0, sem0)

        pltpu.make_async_copy(emb_hbm.at[ids_v.at[ch + 1]], rows1, sem1).wait()
        _accum_stream(rows1, acc_v, ch + 1)

    inv = jnp.full((_LANES,), 1.0 / _L, jnp.float32)
    zero = jnp.zeros((_LANES,), jnp.float32)

    @pl.loop(0, _BPW)
    def _act(i):
        for j in range(_NV):
            v = acc_v[i, pl.ds(j * _LANES, _LANES)]
            acc_v[i, pl.ds(j * _LANES, _LANES)] = jnp.maximum(v * inv, zero)

    pltpu.sync_copy(acc_v, out_hbm.at[w])


_sc_pool = pl.kernel(
    _sc_pool_body,
    out_type=jax.ShapeDtypeStruct((_NW, _BPW, _H), jnp.float32),
    mesh=plsc.VectorSubcoreMesh(core_axis_name="c", subcore_axis_name="s"),
    scratch_types=[
        pltpu.VMEM((_NCH, _ROWS), jnp.int32),
        pltpu.VMEM((_ROWS, _H), jnp.float32),
        pltpu.VMEM((_ROWS, _H), jnp.float32),
        pltpu.VMEM((_BPW, _H), jnp.float32),
        pltpu.SemaphoreType.DMA,
        pltpu.SemaphoreType.DMA,
    ],
)


_MB = 512
_GM = _BC // _MB


def _mm_body_first(x_ref, w_ref, b_ref, o_ref):
    o_ref[...] = lax.dot_general(
        x_ref[...], w_ref[...], (((1,), (1,)), ((), ())),
        preferred_element_type=jnp.float32,
    ) + b_ref[...]


def _mm_body_rest(x_ref, w_ref, b_ref, prev_ref, o_ref):
    del prev_ref
    o_ref[...] = lax.dot_general(
        x_ref[...], w_ref[...], (((1,), (1,)), ((), ())),
        preferred_element_type=jnp.float32,
    ) + b_ref[...]


def _mm_chunk(k, x, W, b2, prev):
    out_spec = pl.BlockSpec((_MB, _O), lambda i, k=k: (k * _GM + i, 0))
    in_specs = [
        pl.BlockSpec((_MB, _H), lambda i: (i, 0)),
        pl.BlockSpec((_O, _H), lambda i: (0, 0)),
        pl.BlockSpec((1, _O), lambda i: (0, 0)),
    ]
    if k == 0:
        return pl.pallas_call(
            _mm_body_first,
            grid=(_GM,),
            in_specs=in_specs,
            out_specs=out_spec,
            out_shape=jax.ShapeDtypeStruct((_B, _O), jnp.float32),
        )(x, W, b2)
    return pl.pallas_call(
        _mm_body_rest,
        grid=(_GM,),
        in_specs=in_specs + [pl.BlockSpec(memory_space=pl.ANY)],
        out_specs=out_spec,
        out_shape=jax.ShapeDtypeStruct((_B, _O), jnp.float32),
        input_output_aliases={3: 0},
    )(x, W, b2, prev)


@jax.jit
def _impl(input_ids, emb, W, b):
    b2 = b.reshape(1, _O)
    ids = input_ids.reshape(_NCHUNK, _NW, _NCH, _ROWS)
    pooled = [_sc_pool(ids[k], emb) for k in range(_NCHUNK)]
    out = None
    for k in range(_NCHUNK):
        out = _mm_chunk(k, pooled[k].reshape(_BC, _H), W, b2, out)
    return out


def kernel(input_ids, token_type_ids, attention_mask, emb, W, b):
    return _impl(input_ids, emb, W, b)


# speedup vs baseline: 1.1140x; 1.0263x over previous
import functools

import numpy as np
import jax
import jax.numpy as jnp
from jax import lax
from jax.experimental import pallas as pl
from jax.experimental.pallas import tpu as pltpu
from jax.experimental.pallas import tpu_sc as plsc

_B, _L, _V, _H, _O = 4096, 50, 100000, 128, 10000
_NCHUNK = 2
_BC = _B // _NCHUNK
_NC, _NS = 2, 16
_NW = _NC * _NS
_BPW = _BC // _NW
_CB = 2
_ROWS = _CB * _L
_NCH = _BPW // _CB
_LANES = 16
_NV = _H // _LANES


def _accum_stream(rows, acc_v, ch):
    for b in range(_CB):
        base = b * _L
        init = tuple(rows[base, pl.ds(j * _LANES, _LANES)] for j in range(_NV))

        @pl.loop(1, _L, init_carry=init, unroll=7)
        def _sum(r, accs):
            return tuple(
                accs[j] + rows[base + r, pl.ds(j * _LANES, _LANES)]
                for j in range(_NV)
            )

        accs = _sum
        for j in range(_NV):
            acc_v[ch * _CB + b, pl.ds(j * _LANES, _LANES)] = accs[j]


def _sc_pool_body(ids_hbm, emb_hbm, out_hbm,
                  ids_v, rows0, rows1, acc_v, sem0, sem1):
    c = lax.axis_index("c")
    s = lax.axis_index("s")
    w = c * _NS + s

    pltpu.sync_copy(ids_hbm.at[w], ids_v)

    pltpu.async_copy(emb_hbm.at[ids_v.at[0]], rows0, sem0)

    @pl.loop(0, _NCH, step=2)
    def _streams(ch):
        pltpu.async_copy(emb_hbm.at[ids_v.at[ch + 1]], rows1, sem1)
        pltpu.make_async_copy(emb_hbm.at[ids_v.at[ch]], rows0, sem0).wait()
        _accum_stream(rows0, acc_v, ch)

        @pl.when(ch + 2 < _NCH)
        def _():
            pltpu.async_copy(emb_hbm.at[ids_v.at[ch + 2]], rows0, sem0)

        pltpu.make_async_copy(emb_hbm.at[ids_v.at[ch + 1]], rows1, sem1).wait()
        _accum_stream(rows1, acc_v, ch + 1)

    inv = jnp.full((_LANES,), 1.0 / _L, jnp.float32)
    zero = jnp.zeros((_LANES,), jnp.float32)

    @pl.loop(0, _BPW)
    def _act(i):
        for j in range(_NV):
            v = acc_v[i, pl.ds(j * _LANES, _LANES)]
            acc_v[i, pl.ds(j * _LANES, _LANES)] = jnp.maximum(v * inv, zero)

    pltpu.sync_copy(acc_v, out_hbm.at[w])


_sc_pool = pl.kernel(
    _sc_pool_body,
    out_type=jax.ShapeDtypeStruct((_NW, _BPW, _H), jnp.float32),
    mesh=plsc.VectorSubcoreMesh(core_axis_name="c", subcore_axis_name="s"),
    scratch_types=[
        pltpu.VMEM((_NCH, _ROWS), jnp.int32),
        pltpu.VMEM((_ROWS, _H), jnp.float32),
        pltpu.VMEM((_ROWS, _H), jnp.float32),
        pltpu.VMEM((_BPW, _H), jnp.float32),
        pltpu.SemaphoreType.DMA,
        pltpu.SemaphoreType.DMA,
    ],
)


_MB = 512
_GM = _BC // _MB


def _mm_body_first(x_ref, w_ref, b_ref, o_ref):
    o_ref[...] = lax.dot_general(
        x_ref[...], w_ref[...], (((1,), (1,)), ((), ())),
        preferred_element_type=jnp.float32,
    ) + b_ref[...]


def _mm_body_rest(x_ref, w_ref, b_ref, prev_ref, o_ref):
    del prev_ref
    o_ref[...] = lax.dot_general(
        x_ref[...], w_ref[...], (((1,), (1,)), ((), ())),
        preferred_element_type=jnp.float32,
    ) + b_ref[...]


def _mm_chunk(k, x, W, b2, prev):
    out_spec = pl.BlockSpec((_MB, _O), lambda i, k=k: (k * _GM + i, 0))
    in_specs = [
        pl.BlockSpec((_MB, _H), lambda i: (i, 0)),
        pl.BlockSpec((_O, _H), lambda i: (0, 0)),
        pl.BlockSpec((1, _O), lambda i: (0, 0)),
    ]
    if k == 0:
        return pl.pallas_call(
            _mm_body_first,
            grid=(_GM,),
            in_specs=in_specs,
            out_specs=out_spec,
            out_shape=jax.ShapeDtypeStruct((_B, _O), jnp.float32),
        )(x, W, b2)
    return pl.pallas_call(
        _mm_body_rest,
        grid=(_GM,),
        in_specs=in_specs + [pl.BlockSpec(memory_space=pl.ANY)],
        out_specs=out_spec,
        out_shape=jax.ShapeDtypeStruct((_B, _O), jnp.float32),
        input_output_aliases={3: 0},
    )(x, W, b2, prev)


@jax.jit
def _impl(input_ids, emb, W, b):
    b2 = b.reshape(1, _O)
    ids = input_ids.reshape(_NCHUNK, _NW, _NCH, _ROWS)
    pooled = [_sc_pool(ids[k], emb) for k in range(_NCHUNK)]
    out = None
    for k in range(_NCHUNK):
        out = _mm_chunk(k, pooled[k].reshape(_BC, _H), W, b2, out)
    return out


def kernel(input_ids, token_type_ids, attention_mask, emb, W, b):
    return _impl(input_ids, emb, W, b)
